# Initial kernel scaffold; baseline (speedup 1.0000x reference)
import jax
import jax.numpy as jnp
from jax.experimental import pallas as pl


def kernel(obj_feats, rel_feats, map_sub, map_obj, W0, b0, W1, b1, W2, b2, W3, b3):
    raise NotImplementedError("write your pallas kernel here")


# speedup vs baseline: 1.9634x; 1.9634x over previous
import jax
import jax.numpy as jnp
from jax.experimental import pallas as pl
from jax.experimental.pallas import tpu as pltpu

N_OBJ = 4096
N_REL = 16384
DIM = 128
TJ = 256
NJ = N_REL // TJ


def _fc_body(rel_ref, obj_ref, w0_ref, b0_ref, w1_ref, b1_ref, w2_ref, b2_ref,
             w3_ref, b3_ref, fc0_ref, fc1_ref, fc2_ref, fc3_ref):
    rel = rel_ref[...]
    obj = obj_ref[...]

    def unit(src, w_ref, b_ref):
        y = jnp.dot(src, w_ref[...], preferred_element_type=jnp.float32) + b_ref[...]
        return jnp.maximum(y, 0.0).astype(jnp.bfloat16)

    fc0_ref[...] = unit(rel, w0_ref, b0_ref)
    fc1_ref[...] = unit(rel, w1_ref, b1_ref)
    fc2_ref[...] = unit(obj, w2_ref, b2_ref)
    fc3_ref[...] = unit(obj, w3_ref, b3_ref)


def _gcn_body(ms_ref, mo_ref, fc0_ref, fc1_ref, fc2_ref, fc3_ref, objf_ref,
              relf_ref, obj_out_ref, rel_out_ref, accs_ref, acco_ref,
              rows_ref, rowo_ref):
    j = pl.program_id(0)
    ms = ms_ref[...]
    mo = mo_ref[...]
    ms_b = ms.astype(jnp.bfloat16)
    mo_b = mo.astype(jnp.bfloat16)

    tdims = (((0,), (0,)), ((), ()))
    cs = jax.lax.dot_general(ms_b, fc2_ref[...], tdims,
                             preferred_element_type=jnp.float32)
    co = jax.lax.dot_general(mo_b, fc3_ref[...], tdims,
                             preferred_element_type=jnp.float32)
    ones = jnp.ones((N_OBJ, 8), dtype=jnp.float32)
    dens = jax.lax.dot_general(ms, ones, tdims,
                               preferred_element_type=jnp.float32)
    deno = jax.lax.dot_general(mo, ones, tdims,
                               preferred_element_type=jnp.float32)
    rel_out_ref[...] = relf_ref[...] + 0.5 * (
        cs / (dens[:, :1] + 1e-7) + co / (deno[:, :1] + 1e-7))

    @pl.when(j == 0)
    def _init():
        accs_ref[...] = jnp.zeros_like(accs_ref)
        acco_ref[...] = jnp.zeros_like(acco_ref)
        rows_ref[...] = jnp.zeros_like(rows_ref)
        rowo_ref[...] = jnp.zeros_like(rowo_ref)

    accs_ref[...] += jnp.dot(ms_b, fc0_ref[...],
                             preferred_element_type=jnp.float32)
    acco_ref[...] += jnp.dot(mo_b, fc1_ref[...],
                             preferred_element_type=jnp.float32)
    rows_ref[...] += jnp.sum(ms, axis=1, keepdims=True)
    rowo_ref[...] += jnp.sum(mo, axis=1, keepdims=True)

    @pl.when(j == NJ - 1)
    def _finish():
        obj_out_ref[...] = objf_ref[...] + 0.5 * (
            accs_ref[...] / (rows_ref[...] + 1e-7)
            + acco_ref[...] / (rowo_ref[...] + 1e-7))


def kernel(obj_feats, rel_feats, map_sub, map_obj, W0, b0, W1, b1, W2, b2, W3, b3):
    fc0, fc1, fc2, fc3 = pl.pallas_call(
        _fc_body,
        out_shape=[
            jax.ShapeDtypeStruct((N_REL, DIM), jnp.bfloat16),
            jax.ShapeDtypeStruct((N_REL, DIM), jnp.bfloat16),
            jax.ShapeDtypeStruct((N_OBJ, DIM), jnp.bfloat16),
            jax.ShapeDtypeStruct((N_OBJ, DIM), jnp.bfloat16),
        ],
    )(rel_feats, obj_feats, W0, b0.reshape(1, DIM), W1, b1.reshape(1, DIM),
      W2, b2.reshape(1, DIM), W3, b3.reshape(1, DIM))

    obj_out, rel_out = pl.pallas_call(
        _gcn_body,
        grid=(NJ,),
        in_specs=[
            pl.BlockSpec((N_OBJ, TJ), lambda j: (0, j)),
            pl.BlockSpec((N_OBJ, TJ), lambda j: (0, j)),
            pl.BlockSpec((TJ, DIM), lambda j: (j, 0)),
            pl.BlockSpec((TJ, DIM), lambda j: (j, 0)),
            pl.BlockSpec((N_OBJ, DIM), lambda j: (0, 0)),
            pl.BlockSpec((N_OBJ, DIM), lambda j: (0, 0)),
            pl.BlockSpec((N_OBJ, DIM), lambda j: (0, 0)),
            pl.BlockSpec((TJ, DIM), lambda j: (j, 0)),
        ],
        out_specs=[
            pl.BlockSpec((N_OBJ, DIM), lambda j: (0, 0)),
            pl.BlockSpec((TJ, DIM), lambda j: (j, 0)),
        ],
        out_shape=[
            jax.ShapeDtypeStruct((N_OBJ, DIM), jnp.float32),
            jax.ShapeDtypeStruct((N_REL, DIM), jnp.float32),
        ],
        scratch_shapes=[
            pltpu.VMEM((N_OBJ, DIM), jnp.float32),
            pltpu.VMEM((N_OBJ, DIM), jnp.float32),
            pltpu.VMEM((N_OBJ, 1), jnp.float32),
            pltpu.VMEM((N_OBJ, 1), jnp.float32),
        ],
    )(map_sub, map_obj, fc0, fc1, fc2, fc3, obj_feats, rel_feats)
    return obj_out, rel_out


# speedup vs baseline: 2.4346x; 1.2399x over previous
import jax
import jax.numpy as jnp
from jax.experimental import pallas as pl
from jax.experimental.pallas import tpu as pltpu

N_OBJ = 4096
N_REL = 16384
DIM = 128
TJ = 256
NJ = N_REL // TJ


def _fc_body(rel_ref, obj_ref, w0_ref, b0_ref, w1_ref, b1_ref, w2_ref, b2_ref,
             w3_ref, b3_ref, fc0_ref, fc1_ref, fc2t_ref, fc3t_ref):
    rel = rel_ref[...]
    obj = obj_ref[...]

    def unit(src, w_ref, b_ref):
        y = jnp.dot(src, w_ref[...], preferred_element_type=jnp.float32) + b_ref[...]
        return jnp.maximum(y, 0.0).astype(jnp.bfloat16)

    fc0_ref[...] = unit(rel, w0_ref, b0_ref)
    fc1_ref[...] = unit(rel, w1_ref, b1_ref)
    fc2t_ref[...] = unit(obj, w2_ref, b2_ref).T
    fc3t_ref[...] = unit(obj, w3_ref, b3_ref).T


def _gcn_body(ms_ref, mo_ref, fc0_ref, fc1_ref, fc2t_ref, fc3t_ref, objf_ref,
              relf_ref, obj_out_ref, rel_out_ref, accs_ref, acco_ref,
              rows_ref, rowo_ref):
    j = pl.program_id(0)
    ms_b = ms_ref[...].astype(jnp.bfloat16)
    mo_b = mo_ref[...].astype(jnp.bfloat16)

    ts = jnp.dot(fc2t_ref[...], ms_b, preferred_element_type=jnp.float32)
    to = jnp.dot(fc3t_ref[...], mo_b, preferred_element_type=jnp.float32)
    onest = jnp.ones((8, N_OBJ), dtype=jnp.bfloat16)
    dens = jnp.dot(onest, ms_b, preferred_element_type=jnp.float32)
    deno = jnp.dot(onest, mo_b, preferred_element_type=jnp.float32)
    r = 0.5 * (ts / (dens[:1, :] + 1e-7) + to / (deno[:1, :] + 1e-7))
    rel_out_ref[...] = relf_ref[...] + r.T

    @pl.when(j == 0)
    def _init():
        accs_ref[...] = jnp.zeros_like(accs_ref)
        acco_ref[...] = jnp.zeros_like(acco_ref)
        rows_ref[...] = jnp.zeros_like(rows_ref)
        rowo_ref[...] = jnp.zeros_like(rowo_ref)

    accs_ref[...] += jnp.dot(ms_b, fc0_ref[...],
                             preferred_element_type=jnp.float32)
    acco_ref[...] += jnp.dot(mo_b, fc1_ref[...],
                             preferred_element_type=jnp.float32)
    rows_ref[...] += jnp.sum(ms_b.astype(jnp.float32), axis=1, keepdims=True)
    rowo_ref[...] += jnp.sum(mo_b.astype(jnp.float32), axis=1, keepdims=True)

    @pl.when(j == NJ - 1)
    def _finish():
        obj_out_ref[...] = objf_ref[...] + 0.5 * (
            accs_ref[...] / (rows_ref[...] + 1e-7)
            + acco_ref[...] / (rowo_ref[...] + 1e-7))


def kernel(obj_feats, rel_feats, map_sub, map_obj, W0, b0, W1, b1, W2, b2, W3, b3):
    fc0, fc1, fc2t, fc3t = pl.pallas_call(
        _fc_body,
        out_shape=[
            jax.ShapeDtypeStruct((N_REL, DIM), jnp.bfloat16),
            jax.ShapeDtypeStruct((N_REL, DIM), jnp.bfloat16),
            jax.ShapeDtypeStruct((DIM, N_OBJ), jnp.bfloat16),
            jax.ShapeDtypeStruct((DIM, N_OBJ), jnp.bfloat16),
        ],
    )(rel_feats, obj_feats, W0, b0.reshape(1, DIM), W1, b1.reshape(1, DIM),
      W2, b2.reshape(1, DIM), W3, b3.reshape(1, DIM))

    obj_out, rel_out = pl.pallas_call(
        _gcn_body,
        grid=(NJ,),
        in_specs=[
            pl.BlockSpec((N_OBJ, TJ), lambda j: (0, j)),
            pl.BlockSpec((N_OBJ, TJ), lambda j: (0, j)),
            pl.BlockSpec((TJ, DIM), lambda j: (j, 0)),
            pl.BlockSpec((TJ, DIM), lambda j: (j, 0)),
            pl.BlockSpec((DIM, N_OBJ), lambda j: (0, 0)),
            pl.BlockSpec((DIM, N_OBJ), lambda j: (0, 0)),
            pl.BlockSpec((N_OBJ, DIM), lambda j: (0, 0)),
            pl.BlockSpec((TJ, DIM), lambda j: (j, 0)),
        ],
        out_specs=[
            pl.BlockSpec((N_OBJ, DIM), lambda j: (0, 0)),
            pl.BlockSpec((TJ, DIM), lambda j: (j, 0)),
        ],
        out_shape=[
            jax.ShapeDtypeStruct((N_OBJ, DIM), jnp.float32),
            jax.ShapeDtypeStruct((N_REL, DIM), jnp.float32),
        ],
        scratch_shapes=[
            pltpu.VMEM((N_OBJ, DIM), jnp.float32),
            pltpu.VMEM((N_OBJ, DIM), jnp.float32),
            pltpu.VMEM((N_OBJ, 1), jnp.float32),
            pltpu.VMEM((N_OBJ, 1), jnp.float32),
        ],
    )(map_sub, map_obj, fc0, fc1, fc2t, fc3t, obj_feats, rel_feats)
    return obj_out, rel_out


# speedup vs baseline: 2.8318x; 1.1632x over previous
import jax
import jax.numpy as jnp
from jax.experimental import pallas as pl
from jax.experimental.pallas import tpu as pltpu

N_OBJ = 4096
N_REL = 16384
DIM = 128
TJ = 512
NJ = N_REL // TJ


def _fc_body(rel_ref, obj_ref, w0_ref, b0_ref, w1_ref, b1_ref, w2_ref, b2_ref,
             w3_ref, b3_ref, fc0_ref, fc1_ref, fc2t_ref, fc3t_ref):
    rel = rel_ref[...]
    obj = obj_ref[...]

    def unit(src, w_ref, b_ref):
        y = jnp.dot(src, w_ref[...], preferred_element_type=jnp.float32) + b_ref[...]
        return jnp.maximum(y, 0.0).astype(jnp.bfloat16)

    fc0_ref[...] = unit(rel, w0_ref, b0_ref)
    fc1_ref[...] = unit(rel, w1_ref, b1_ref)
    fc2t_ref[:DIM, :] = unit(obj, w2_ref, b2_ref).T
    fc2t_ref[DIM:, :] = jnp.ones((8, N_OBJ), jnp.bfloat16)
    fc3t_ref[:DIM, :] = unit(obj, w3_ref, b3_ref).T
    fc3t_ref[DIM:, :] = jnp.ones((8, N_OBJ), jnp.bfloat16)


def _gcn_body(ms_ref, mo_ref, fc0_ref, fc1_ref, fc2t_ref, fc3t_ref, objf_ref,
              relf_ref, obj_out_ref, rel_out_ref, accs_ref, acco_ref,
              rows_ref, rowo_ref):
    j = pl.program_id(0)
    ms_b = ms_ref[...].astype(jnp.bfloat16)
    mo_b = mo_ref[...].astype(jnp.bfloat16)

    tsa = jnp.dot(fc2t_ref[...], ms_b, preferred_element_type=jnp.float32)
    toa = jnp.dot(fc3t_ref[...], mo_b, preferred_element_type=jnp.float32)
    r = 0.5 * (tsa[:DIM, :] / (tsa[DIM:DIM + 1, :] + 1e-7)
               + toa[:DIM, :] / (toa[DIM:DIM + 1, :] + 1e-7))
    rel_out_ref[...] = relf_ref[...] + r.T

    @pl.when(j == 0)
    def _init():
        accs_ref[...] = jnp.zeros_like(accs_ref)
        acco_ref[...] = jnp.zeros_like(acco_ref)
        rows_ref[...] = jnp.zeros_like(rows_ref)
        rowo_ref[...] = jnp.zeros_like(rowo_ref)

    accs_ref[...] += jnp.dot(ms_b, fc0_ref[...],
                             preferred_element_type=jnp.float32)
    acco_ref[...] += jnp.dot(mo_b, fc1_ref[...],
                             preferred_element_type=jnp.float32)
    rows_ref[...] += jnp.sum(ms_b.astype(jnp.float32), axis=1, keepdims=True)
    rowo_ref[...] += jnp.sum(mo_b.astype(jnp.float32), axis=1, keepdims=True)

    @pl.when(j == NJ - 1)
    def _finish():
        obj_out_ref[...] = objf_ref[...] + 0.5 * (
            accs_ref[...] / (rows_ref[...] + 1e-7)
            + acco_ref[...] / (rowo_ref[...] + 1e-7))


def kernel(obj_feats, rel_feats, map_sub, map_obj, W0, b0, W1, b1, W2, b2, W3, b3):
    fc0, fc1, fc2t, fc3t = pl.pallas_call(
        _fc_body,
        out_shape=[
            jax.ShapeDtypeStruct((N_REL, DIM), jnp.bfloat16),
            jax.ShapeDtypeStruct((N_REL, DIM), jnp.bfloat16),
            jax.ShapeDtypeStruct((DIM + 8, N_OBJ), jnp.bfloat16),
            jax.ShapeDtypeStruct((DIM + 8, N_OBJ), jnp.bfloat16),
        ],
    )(rel_feats, obj_feats, W0, b0.reshape(1, DIM), W1, b1.reshape(1, DIM),
      W2, b2.reshape(1, DIM), W3, b3.reshape(1, DIM))

    obj_out, rel_out = pl.pallas_call(
        _gcn_body,
        grid=(NJ,),
        in_specs=[
            pl.BlockSpec((N_OBJ, TJ), lambda j: (0, j)),
            pl.BlockSpec((N_OBJ, TJ), lambda j: (0, j)),
            pl.BlockSpec((TJ, DIM), lambda j: (j, 0)),
            pl.BlockSpec((TJ, DIM), lambda j: (j, 0)),
            pl.BlockSpec((DIM + 8, N_OBJ), lambda j: (0, 0)),
            pl.BlockSpec((DIM + 8, N_OBJ), lambda j: (0, 0)),
            pl.BlockSpec((N_OBJ, DIM), lambda j: (0, 0)),
            pl.BlockSpec((TJ, DIM), lambda j: (j, 0)),
        ],
        out_specs=[
            pl.BlockSpec((N_OBJ, DIM), lambda j: (0, 0)),
            pl.BlockSpec((TJ, DIM), lambda j: (j, 0)),
        ],
        out_shape=[
            jax.ShapeDtypeStruct((N_OBJ, DIM), jnp.float32),
            jax.ShapeDtypeStruct((N_REL, DIM), jnp.float32),
        ],
        scratch_shapes=[
            pltpu.VMEM((N_OBJ, DIM), jnp.float32),
            pltpu.VMEM((N_OBJ, DIM), jnp.float32),
            pltpu.VMEM((N_OBJ, 1), jnp.float32),
            pltpu.VMEM((N_OBJ, 1), jnp.float32),
        ],
    )(map_sub, map_obj, fc0, fc1, fc2t, fc3t, obj_feats, rel_feats)
    return obj_out, rel_out


# speedup vs baseline: 2.9214x; 1.0317x over previous
import jax
import jax.numpy as jnp
from jax.experimental import pallas as pl
from jax.experimental.pallas import tpu as pltpu

N_OBJ = 4096
N_REL = 16384
DIM = 128
TJ = 512
NJ = N_REL // TJ


def _gcn_body(ms_ref, mo_ref, objf_ref, relf_ref,
              w0_ref, b0_ref, w1_ref, b1_ref, w2_ref, b2_ref, w3_ref, b3_ref,
              obj_out_ref, rel_out_ref,
              accs_ref, acco_ref, rows_ref, rowo_ref, fc2t_ref, fc3t_ref):
    j = pl.program_id(0)

    def unit(src, w_ref, b_ref):
        y = jnp.dot(src, w_ref[...], preferred_element_type=jnp.float32) + b_ref[...]
        return jnp.maximum(y, 0.0).astype(jnp.bfloat16)

    @pl.when(j == 0)
    def _init():
        fc2t_ref[:DIM, :] = unit(objf_ref[...], w2_ref, b2_ref).T
        fc2t_ref[DIM:, :] = jnp.ones((8, N_OBJ), jnp.bfloat16)
        fc3t_ref[:DIM, :] = unit(objf_ref[...], w3_ref, b3_ref).T
        fc3t_ref[DIM:, :] = jnp.ones((8, N_OBJ), jnp.bfloat16)
        accs_ref[...] = jnp.zeros_like(accs_ref)
        acco_ref[...] = jnp.zeros_like(acco_ref)
        rows_ref[...] = jnp.zeros_like(rows_ref)
        rowo_ref[...] = jnp.zeros_like(rowo_ref)

    ms_b = ms_ref[...].astype(jnp.bfloat16)
    mo_b = mo_ref[...].astype(jnp.bfloat16)

    tsa = jnp.dot(fc2t_ref[...], ms_b, preferred_element_type=jnp.float32)
    toa = jnp.dot(fc3t_ref[...], mo_b, preferred_element_type=jnp.float32)
    r = 0.5 * (tsa[:DIM, :] / (tsa[DIM:DIM + 1, :] + 1e-7)
               + toa[:DIM, :] / (toa[DIM:DIM + 1, :] + 1e-7))
    rel_out_ref[...] = relf_ref[...] + r.T

    accs_ref[...] += jnp.dot(ms_b, unit(relf_ref[...], w0_ref, b0_ref),
                             preferred_element_type=jnp.float32)
    acco_ref[...] += jnp.dot(mo_b, unit(relf_ref[...], w1_ref, b1_ref),
                             preferred_element_type=jnp.float32)
    rows_ref[...] += jnp.sum(ms_b.astype(jnp.float32), axis=1, keepdims=True)
    rowo_ref[...] += jnp.sum(mo_b.astype(jnp.float32), axis=1, keepdims=True)

    @pl.when(j == NJ - 1)
    def _finish():
        obj_out_ref[...] = objf_ref[...] + 0.5 * (
            accs_ref[...] / (rows_ref[...] + 1e-7)
            + acco_ref[...] / (rowo_ref[...] + 1e-7))


def kernel(obj_feats, rel_feats, map_sub, map_obj, W0, b0, W1, b1, W2, b2, W3, b3):
    const = pl.BlockSpec((DIM, DIM), lambda j: (0, 0))
    bconst = pl.BlockSpec((1, DIM), lambda j: (0, 0))
    obj_out, rel_out = pl.pallas_call(
        _gcn_body,
        grid=(NJ,),
        in_specs=[
            pl.BlockSpec((N_OBJ, TJ), lambda j: (0, j)),
            pl.BlockSpec((N_OBJ, TJ), lambda j: (0, j)),
            pl.BlockSpec((N_OBJ, DIM), lambda j: (0, 0)),
            pl.BlockSpec((TJ, DIM), lambda j: (j, 0)),
            const, bconst, const, bconst, const, bconst, const, bconst,
        ],
        out_specs=[
            pl.BlockSpec((N_OBJ, DIM), lambda j: (0, 0)),
            pl.BlockSpec((TJ, DIM), lambda j: (j, 0)),
        ],
        out_shape=[
            jax.ShapeDtypeStruct((N_OBJ, DIM), jnp.float32),
            jax.ShapeDtypeStruct((N_REL, DIM), jnp.float32),
        ],
        scratch_shapes=[
            pltpu.VMEM((N_OBJ, DIM), jnp.float32),
            pltpu.VMEM((N_OBJ, DIM), jnp.float32),
            pltpu.VMEM((N_OBJ, 1), jnp.float32),
            pltpu.VMEM((N_OBJ, 1), jnp.float32),
            pltpu.VMEM((DIM + 8, N_OBJ), jnp.bfloat16),
            pltpu.VMEM((DIM + 8, N_OBJ), jnp.bfloat16),
        ],
    )(map_sub, map_obj, obj_feats, rel_feats,
      W0, b0.reshape(1, DIM), W1, b1.reshape(1, DIM),
      W2, b2.reshape(1, DIM), W3, b3.reshape(1, DIM))
    return obj_out, rel_out
